# baseline (speedup 1.0000x reference)
import jax
import jax.numpy as jnp
from jax import lax
from jax.experimental import pallas as pl
from jax.experimental.pallas import tpu as pltpu

N = 8192
NUM_SLICES = 16
K = 20
BLK = 256
CW = 512
NBLK = N // BLK
BESTW = 128
D_OUT = 256

_INF = float("inf")
_BIG_I = 2**30


def _body(clo_ref, chi_ref, xr_ref, br_ref, xc_ref, bc_ref,
          wi0_ref, wi1_ref, wj0_ref, wj1_ref,
          b1_ref, w2_ref, b2_ref, w3_ref, b3_ref, out_ref):
    i = pl.program_id(0)
    xr = xr_ref[...]
    br = br_ref[...]
    sq_r = jnp.sum(xr * xr, axis=1, keepdims=True)

    row_iota = lax.broadcasted_iota(jnp.int32, (BLK, CW), 0)
    col_iota = lax.broadcasted_iota(jnp.int32, (BLK, CW), 1)
    iota_w = lax.broadcasted_iota(
        jnp.int32, (BLK, BESTW + CW), 1).astype(jnp.float32)

    def chunk_body(c, carry):
        best_key, best_x0, best_x1 = carry
        cs = c * CW
        xw = xc_ref[:, pl.ds(cs, CW)]
        bw = bc_ref[:, pl.ds(cs, CW)]
        xw0 = xw[0:1, :]
        xw1 = xw[1:2, :]
        sq_w = xw0 * xw0 + xw1 * xw1
        dot = jnp.dot(xr, xw, preferred_element_type=jnp.float32)
        d2 = (sq_r + sq_w) - 2.0 * dot
        same = br == bw
        d2 = jnp.where(same, d2, _INF)
        rowg = i * BLK + row_iota
        colg = cs + col_iota
        d2 = jnp.where(rowg == colg, _INF, d2)

        wa = jnp.concatenate([best_key, d2], axis=1)

        ks, x0s, x1s = [], [], []
        for _ in range(K):
            m = jnp.min(wa, axis=1, keepdims=True)
            pos = jnp.min(jnp.where(wa == m, iota_w, _INF),
                          axis=1, keepdims=True)
            hit = iota_w == pos
            hitb = hit[:, :BESTW]
            hitc = hit[:, BESTW:]
            hcf = jnp.where(hitc, 1.0, 0.0)
            xpair = lax.dot_general(
                hcf, xw, (((1,), (1,)), ((), ())),
                precision=lax.Precision.HIGHEST,
                preferred_element_type=jnp.float32)
            x0k = xpair[:, 0:1] + jnp.sum(jnp.where(hitb, best_x0, 0.0),
                                          axis=1, keepdims=True)
            x1k = xpair[:, 1:2] + jnp.sum(jnp.where(hitb, best_x1, 0.0),
                                          axis=1, keepdims=True)
            wa = jnp.where(hit, _INF, wa)
            ks.append(m)
            x0s.append(x0k)
            x1s.append(x1k)

        pad = jnp.full((BLK, BESTW - K), _INF, jnp.float32)
        zpad = jnp.zeros((BLK, BESTW - K), jnp.float32)
        nbk = jnp.concatenate(ks + [pad], axis=1)
        nb0 = jnp.concatenate(x0s + [zpad], axis=1)
        nb1 = jnp.concatenate(x1s + [zpad], axis=1)
        return nbk, nb0, nb1

    init = (jnp.full((BLK, BESTW), _INF, jnp.float32),
            jnp.zeros((BLK, BESTW), jnp.float32),
            jnp.zeros((BLK, BESTW), jnp.float32))
    c0 = clo_ref[i]
    c1 = chi_ref[i]
    _, bx0, bx1 = lax.fori_loop(c0, c1, chunk_body, init)

    xr0 = xr[:, 0:1]
    xr1 = xr[:, 1:2]
    a = xr0 * wi0_ref[...] + xr1 * wi1_ref[...] + b1_ref[...]
    hs = []
    for k in range(K):
        cjk = bx0[:, k:k + 1] * wj0_ref[...] + bx1[:, k:k + 1] * wj1_ref[...]
        hs.append(jnp.maximum(a + cjk, 0.0))
    h = jnp.concatenate(hs, axis=0)

    h = jnp.maximum(jnp.dot(h, w2_ref[...],
                            preferred_element_type=jnp.float32) + b2_ref[...], 0.0)
    msg = jnp.dot(h, w3_ref[...],
                  preferred_element_type=jnp.float32) + b3_ref[...]

    pf = msg[0:BLK, :]
    for k in range(1, K):
        pf = jnp.maximum(pf, msg[k * BLK:(k + 1) * BLK, :])

    rows = []
    for s in range(NUM_SLICES):
        sel = br == s
        rows.append(jnp.max(jnp.where(sel, pf, -_INF), axis=0, keepdims=True))
    blk_out = jnp.concatenate(rows, axis=0)

    @pl.when(i == 0)
    def _():
        out_ref[...] = jnp.full((NUM_SLICES, D_OUT), -_INF, jnp.float32)

    out_ref[...] = jnp.maximum(out_ref[...], blk_out)


def kernel(x, batch, W1, b1, W2, b2, W3, b3):
    batch_i = batch.astype(jnp.int32)
    xc = x.T
    br = batch_i[:, None]
    bc = batch_i[None, :]
    sl = jnp.arange(NUM_SLICES, dtype=jnp.int32)
    starts = jnp.searchsorted(batch_i, sl, side="left").astype(jnp.int32)
    ends = jnp.searchsorted(batch_i, sl, side="right").astype(jnp.int32)
    first_b = batch_i[0::BLK]
    last_b = batch_i[BLK - 1::BLK]
    clo = (starts[first_b] // CW).astype(jnp.int32)
    chi = ((ends[last_b] + CW - 1) // CW).astype(jnp.int32)
    wi0 = (W1[0] - W1[2])[None, :]
    wi1 = (W1[1] - W1[3])[None, :]
    wj0 = W1[2][None, :]
    wj1 = W1[3][None, :]

    smem = pl.BlockSpec(memory_space=pltpu.SMEM)
    full = lambda shape: pl.BlockSpec(shape, lambda i: (0, 0))
    out = pl.pallas_call(
        _body,
        grid=(NBLK,),
        in_specs=[
            smem, smem,
            pl.BlockSpec((BLK, 2), lambda i: (i, 0)),
            pl.BlockSpec((BLK, 1), lambda i: (i, 0)),
            full((2, N)),
            full((1, N)),
            full((1, 64)),
            full((1, 64)),
            full((1, 64)),
            full((1, 64)),
            full((1, 64)),
            full((64, 128)),
            full((1, 128)),
            full((128, 256)),
            full((1, 256)),
        ],
        out_specs=pl.BlockSpec((NUM_SLICES, D_OUT), lambda i: (0, 0)),
        out_shape=jax.ShapeDtypeStruct((NUM_SLICES, D_OUT), jnp.float32),
    )(clo, chi, x, br, xc, bc, wi0, wi1, wj0, wj1,
      b1[None, :], W2, b2[None, :], W3, b3[None, :])
    return out


# speedup vs baseline: 1.4817x; 1.4817x over previous
import jax
import jax.numpy as jnp
from jax import lax
from jax.experimental import pallas as pl
from jax.experimental.pallas import tpu as pltpu

N = 8192
NUM_SLICES = 16
K = 20
BLK = 256
CW = 256
NBLK = N // BLK
BESTW = 128
D_OUT = 256

_INF = float("inf")
_BIG_I = 2**30


def _body(clo_ref, chi_ref, xr_ref, br_ref, xc_ref, bc_ref,
          wi0_ref, wi1_ref, wj0_ref, wj1_ref,
          b1_ref, w2_ref, b2_ref, w3_ref, b3_ref, out_ref):
    i = pl.program_id(0)
    xr = xr_ref[...]
    br = br_ref[...]
    sq_r = jnp.sum(xr * xr, axis=1, keepdims=True)

    row_iota = lax.broadcasted_iota(jnp.int32, (BLK, CW), 0)
    col_iota = lax.broadcasted_iota(jnp.int32, (BLK, CW), 1)
    iota_w = lax.broadcasted_iota(
        jnp.int32, (BLK, BESTW + CW), 1).astype(jnp.float32)

    def chunk_body(c, carry):
        best_key, best_x0, best_x1 = carry
        cs = c * CW
        xw = xc_ref[:, pl.ds(cs, CW)]
        bw = bc_ref[:, pl.ds(cs, CW)]
        xw0 = xw[0:1, :]
        xw1 = xw[1:2, :]
        sq_w = xw0 * xw0 + xw1 * xw1
        dot = jnp.dot(xr, xw, preferred_element_type=jnp.float32)
        d2 = (sq_r + sq_w) - 2.0 * dot
        same = br == bw
        d2 = jnp.where(same, d2, _INF)
        rowg = i * BLK + row_iota
        colg = cs + col_iota
        d2 = jnp.where(rowg == colg, _INF, d2)

        wa = jnp.concatenate([best_key, d2], axis=1)
        wx0 = jnp.concatenate(
            [best_x0, jnp.broadcast_to(xw0, (BLK, CW))], axis=1)
        wx1 = jnp.concatenate(
            [best_x1, jnp.broadcast_to(xw1, (BLK, CW))], axis=1)

        ks, x0s, x1s = [], [], []
        for _ in range(K):
            m = jnp.min(wa, axis=1, keepdims=True)
            pos = jnp.min(jnp.where(wa == m, iota_w, _INF),
                          axis=1, keepdims=True)
            hit = iota_w == pos
            x0k = jnp.sum(jnp.where(hit, wx0, 0.0), axis=1, keepdims=True)
            x1k = jnp.sum(jnp.where(hit, wx1, 0.0), axis=1, keepdims=True)
            wa = jnp.where(hit, _INF, wa)
            ks.append(m)
            x0s.append(x0k)
            x1s.append(x1k)

        pad = jnp.full((BLK, BESTW - K), _INF, jnp.float32)
        zpad = jnp.zeros((BLK, BESTW - K), jnp.float32)
        nbk = jnp.concatenate(ks + [pad], axis=1)
        nb0 = jnp.concatenate(x0s + [zpad], axis=1)
        nb1 = jnp.concatenate(x1s + [zpad], axis=1)
        return nbk, nb0, nb1

    init = (jnp.full((BLK, BESTW), _INF, jnp.float32),
            jnp.zeros((BLK, BESTW), jnp.float32),
            jnp.zeros((BLK, BESTW), jnp.float32))
    c0 = clo_ref[i]
    c1 = chi_ref[i]
    _, bx0, bx1 = lax.fori_loop(c0, c1, chunk_body, init)

    xr0 = xr[:, 0:1]
    xr1 = xr[:, 1:2]
    a = xr0 * wi0_ref[...] + xr1 * wi1_ref[...] + b1_ref[...]
    hs = []
    for k in range(K):
        cjk = bx0[:, k:k + 1] * wj0_ref[...] + bx1[:, k:k + 1] * wj1_ref[...]
        hs.append(jnp.maximum(a + cjk, 0.0))
    h = jnp.concatenate(hs, axis=0)

    h = jnp.maximum(jnp.dot(h, w2_ref[...],
                            preferred_element_type=jnp.float32) + b2_ref[...], 0.0)
    msg = jnp.dot(h, w3_ref[...],
                  preferred_element_type=jnp.float32) + b3_ref[...]

    pf = msg[0:BLK, :]
    for k in range(1, K):
        pf = jnp.maximum(pf, msg[k * BLK:(k + 1) * BLK, :])

    rows = []
    for s in range(NUM_SLICES):
        sel = br == s
        rows.append(jnp.max(jnp.where(sel, pf, -_INF), axis=0, keepdims=True))
    blk_out = jnp.concatenate(rows, axis=0)

    @pl.when(i == 0)
    def _():
        out_ref[...] = jnp.full((NUM_SLICES, D_OUT), -_INF, jnp.float32)

    out_ref[...] = jnp.maximum(out_ref[...], blk_out)


def kernel(x, batch, W1, b1, W2, b2, W3, b3):
    batch_i = batch.astype(jnp.int32)
    xc = x.T
    br = batch_i[:, None]
    bc = batch_i[None, :]
    sl = jnp.arange(NUM_SLICES, dtype=jnp.int32)
    starts = jnp.searchsorted(batch_i, sl, side="left").astype(jnp.int32)
    ends = jnp.searchsorted(batch_i, sl, side="right").astype(jnp.int32)
    first_b = batch_i[0::BLK]
    last_b = batch_i[BLK - 1::BLK]
    clo = (starts[first_b] // CW).astype(jnp.int32)
    chi = ((ends[last_b] + CW - 1) // CW).astype(jnp.int32)
    wi0 = (W1[0] - W1[2])[None, :]
    wi1 = (W1[1] - W1[3])[None, :]
    wj0 = W1[2][None, :]
    wj1 = W1[3][None, :]

    smem = pl.BlockSpec(memory_space=pltpu.SMEM)
    full = lambda shape: pl.BlockSpec(shape, lambda i: (0, 0))
    out = pl.pallas_call(
        _body,
        grid=(NBLK,),
        in_specs=[
            smem, smem,
            pl.BlockSpec((BLK, 2), lambda i: (i, 0)),
            pl.BlockSpec((BLK, 1), lambda i: (i, 0)),
            full((2, N)),
            full((1, N)),
            full((1, 64)),
            full((1, 64)),
            full((1, 64)),
            full((1, 64)),
            full((1, 64)),
            full((64, 128)),
            full((1, 128)),
            full((128, 256)),
            full((1, 256)),
        ],
        out_specs=pl.BlockSpec((NUM_SLICES, D_OUT), lambda i: (0, 0)),
        out_shape=jax.ShapeDtypeStruct((NUM_SLICES, D_OUT), jnp.float32),
    )(clo, chi, x, br, xc, bc, wi0, wi1, wj0, wj1,
      b1[None, :], W2, b2[None, :], W3, b3[None, :])
    return out


# speedup vs baseline: 1.4876x; 1.0040x over previous
import jax
import jax.numpy as jnp
from jax import lax
from jax.experimental import pallas as pl
from jax.experimental.pallas import tpu as pltpu

N = 8192
NUM_SLICES = 16
K = 20
BLK = 512
CW = 512
NBLK = N // BLK
BESTW = 128
D_OUT = 256

_INF = float("inf")
_BIG_I = 2**30


def _body(clo_ref, chi_ref, xr_ref, br_ref, xc_ref, bc_ref,
          wi0_ref, wi1_ref, wj0_ref, wj1_ref,
          b1_ref, w2_ref, b2_ref, w3_ref, b3_ref, out_ref):
    i = pl.program_id(0)
    xr = xr_ref[...]
    br = br_ref[...]
    sq_r = jnp.sum(xr * xr, axis=1, keepdims=True)

    row_iota = lax.broadcasted_iota(jnp.int32, (BLK, CW), 0)
    col_iota = lax.broadcasted_iota(jnp.int32, (BLK, CW), 1)
    iota_w = lax.broadcasted_iota(
        jnp.int32, (BLK, BESTW + CW), 1).astype(jnp.float32)

    def chunk_body(c, carry):
        best_key, best_x0, best_x1 = carry
        cs = c * CW
        xw = xc_ref[:, pl.ds(cs, CW)]
        bw = bc_ref[:, pl.ds(cs, CW)]
        xw0 = xw[0:1, :]
        xw1 = xw[1:2, :]
        sq_w = xw0 * xw0 + xw1 * xw1
        dot = jnp.dot(xr, xw, preferred_element_type=jnp.float32)
        d2 = (sq_r + sq_w) - 2.0 * dot
        same = br == bw
        d2 = jnp.where(same, d2, _INF)
        rowg = i * BLK + row_iota
        colg = cs + col_iota
        d2 = jnp.where(rowg == colg, _INF, d2)

        wa = jnp.concatenate([best_key, d2], axis=1)
        wx0 = jnp.concatenate(
            [best_x0, jnp.broadcast_to(xw0, (BLK, CW))], axis=1)
        wx1 = jnp.concatenate(
            [best_x1, jnp.broadcast_to(xw1, (BLK, CW))], axis=1)

        ks, x0s, x1s = [], [], []
        for _ in range(K):
            m = jnp.min(wa, axis=1, keepdims=True)
            pos = jnp.min(jnp.where(wa == m, iota_w, _INF),
                          axis=1, keepdims=True)
            hit = iota_w == pos
            x0k = jnp.sum(jnp.where(hit, wx0, 0.0), axis=1, keepdims=True)
            x1k = jnp.sum(jnp.where(hit, wx1, 0.0), axis=1, keepdims=True)
            wa = jnp.where(hit, _INF, wa)
            ks.append(m)
            x0s.append(x0k)
            x1s.append(x1k)

        pad = jnp.full((BLK, BESTW - K), _INF, jnp.float32)
        zpad = jnp.zeros((BLK, BESTW - K), jnp.float32)
        nbk = jnp.concatenate(ks + [pad], axis=1)
        nb0 = jnp.concatenate(x0s + [zpad], axis=1)
        nb1 = jnp.concatenate(x1s + [zpad], axis=1)
        return nbk, nb0, nb1

    init = (jnp.full((BLK, BESTW), _INF, jnp.float32),
            jnp.zeros((BLK, BESTW), jnp.float32),
            jnp.zeros((BLK, BESTW), jnp.float32))
    c0 = clo_ref[i]
    c1 = chi_ref[i]
    _, bx0, bx1 = lax.fori_loop(c0, c1, chunk_body, init)

    xr0 = xr[:, 0:1]
    xr1 = xr[:, 1:2]
    a = xr0 * wi0_ref[...] + xr1 * wi1_ref[...] + b1_ref[...]
    hs = []
    for k in range(K):
        cjk = bx0[:, k:k + 1] * wj0_ref[...] + bx1[:, k:k + 1] * wj1_ref[...]
        hs.append(jnp.maximum(a + cjk, 0.0))
    h = jnp.concatenate(hs, axis=0)

    h = jnp.maximum(jnp.dot(h, w2_ref[...],
                            preferred_element_type=jnp.float32) + b2_ref[...], 0.0)
    msg = jnp.dot(h, w3_ref[...],
                  preferred_element_type=jnp.float32) + b3_ref[...]

    pf = msg[0:BLK, :]
    for k in range(1, K):
        pf = jnp.maximum(pf, msg[k * BLK:(k + 1) * BLK, :])

    rows = []
    for s in range(NUM_SLICES):
        sel = br == s
        rows.append(jnp.max(jnp.where(sel, pf, -_INF), axis=0, keepdims=True))
    blk_out = jnp.concatenate(rows, axis=0)

    @pl.when(i == 0)
    def _():
        out_ref[...] = jnp.full((NUM_SLICES, D_OUT), -_INF, jnp.float32)

    out_ref[...] = jnp.maximum(out_ref[...], blk_out)


def kernel(x, batch, W1, b1, W2, b2, W3, b3):
    batch_i = batch.astype(jnp.int32)
    xc = x.T
    br = batch_i[:, None]
    bc = batch_i[None, :]
    sl = jnp.arange(NUM_SLICES, dtype=jnp.int32)
    starts = jnp.searchsorted(batch_i, sl, side="left").astype(jnp.int32)
    ends = jnp.searchsorted(batch_i, sl, side="right").astype(jnp.int32)
    first_b = batch_i[0::BLK]
    last_b = batch_i[BLK - 1::BLK]
    clo = (starts[first_b] // CW).astype(jnp.int32)
    chi = ((ends[last_b] + CW - 1) // CW).astype(jnp.int32)
    wi0 = (W1[0] - W1[2])[None, :]
    wi1 = (W1[1] - W1[3])[None, :]
    wj0 = W1[2][None, :]
    wj1 = W1[3][None, :]

    smem = pl.BlockSpec(memory_space=pltpu.SMEM)
    full = lambda shape: pl.BlockSpec(shape, lambda i: (0, 0))
    out = pl.pallas_call(
        _body,
        grid=(NBLK,),
        in_specs=[
            smem, smem,
            pl.BlockSpec((BLK, 2), lambda i: (i, 0)),
            pl.BlockSpec((BLK, 1), lambda i: (i, 0)),
            full((2, N)),
            full((1, N)),
            full((1, 64)),
            full((1, 64)),
            full((1, 64)),
            full((1, 64)),
            full((1, 64)),
            full((64, 128)),
            full((1, 128)),
            full((128, 256)),
            full((1, 256)),
        ],
        out_specs=pl.BlockSpec((NUM_SLICES, D_OUT), lambda i: (0, 0)),
        out_shape=jax.ShapeDtypeStruct((NUM_SLICES, D_OUT), jnp.float32),
    )(clo, chi, x, br, xc, bc, wi0, wi1, wj0, wj1,
      b1[None, :], W2, b2[None, :], W3, b3[None, :])
    return out


# speedup vs baseline: 1.4900x; 1.0016x over previous
import jax
import jax.numpy as jnp
from jax import lax
from jax.experimental import pallas as pl
from jax.experimental.pallas import tpu as pltpu

N = 8192
NUM_SLICES = 16
K = 20
BLK = 128
CW = 512
NBLK = N // BLK
BESTW = 128
D_OUT = 256

_INF = float("inf")
_BIG_I = 2**30


def _body(clo_ref, chi_ref, xr_ref, br_ref, xc_ref, bc_ref,
          wi0_ref, wi1_ref, wj0_ref, wj1_ref,
          b1_ref, w2_ref, b2_ref, w3_ref, b3_ref, out_ref):
    i = pl.program_id(0)
    xr = xr_ref[...]
    br = br_ref[...]
    sq_r = jnp.sum(xr * xr, axis=1, keepdims=True)

    row_iota = lax.broadcasted_iota(jnp.int32, (BLK, CW), 0)
    col_iota = lax.broadcasted_iota(jnp.int32, (BLK, CW), 1)
    iota_w = lax.broadcasted_iota(
        jnp.int32, (BLK, BESTW + CW), 1).astype(jnp.float32)

    def chunk_body(c, carry):
        best_key, best_x0, best_x1 = carry
        cs = c * CW
        xw = xc_ref[:, pl.ds(cs, CW)]
        bw = bc_ref[:, pl.ds(cs, CW)]
        xw0 = xw[0:1, :]
        xw1 = xw[1:2, :]
        sq_w = xw0 * xw0 + xw1 * xw1
        dot = jnp.dot(xr, xw, preferred_element_type=jnp.float32)
        d2 = (sq_r + sq_w) - 2.0 * dot
        same = br == bw
        d2 = jnp.where(same, d2, _INF)
        rowg = i * BLK + row_iota
        colg = cs + col_iota
        d2 = jnp.where(rowg == colg, _INF, d2)

        wa = jnp.concatenate([best_key, d2], axis=1)
        wx0 = jnp.concatenate(
            [best_x0, jnp.broadcast_to(xw0, (BLK, CW))], axis=1)
        wx1 = jnp.concatenate(
            [best_x1, jnp.broadcast_to(xw1, (BLK, CW))], axis=1)

        ks, x0s, x1s = [], [], []
        for _ in range(K):
            m = jnp.min(wa, axis=1, keepdims=True)
            pos = jnp.min(jnp.where(wa == m, iota_w, _INF),
                          axis=1, keepdims=True)
            hit = iota_w == pos
            x0k = jnp.sum(jnp.where(hit, wx0, 0.0), axis=1, keepdims=True)
            x1k = jnp.sum(jnp.where(hit, wx1, 0.0), axis=1, keepdims=True)
            wa = jnp.where(hit, _INF, wa)
            ks.append(m)
            x0s.append(x0k)
            x1s.append(x1k)

        pad = jnp.full((BLK, BESTW - K), _INF, jnp.float32)
        zpad = jnp.zeros((BLK, BESTW - K), jnp.float32)
        nbk = jnp.concatenate(ks + [pad], axis=1)
        nb0 = jnp.concatenate(x0s + [zpad], axis=1)
        nb1 = jnp.concatenate(x1s + [zpad], axis=1)
        return nbk, nb0, nb1

    init = (jnp.full((BLK, BESTW), _INF, jnp.float32),
            jnp.zeros((BLK, BESTW), jnp.float32),
            jnp.zeros((BLK, BESTW), jnp.float32))
    c0 = clo_ref[i]
    c1 = chi_ref[i]
    _, bx0, bx1 = lax.fori_loop(c0, c1, chunk_body, init)

    xr0 = xr[:, 0:1]
    xr1 = xr[:, 1:2]
    a = xr0 * wi0_ref[...] + xr1 * wi1_ref[...] + b1_ref[...]
    hs = []
    for k in range(K):
        cjk = bx0[:, k:k + 1] * wj0_ref[...] + bx1[:, k:k + 1] * wj1_ref[...]
        hs.append(jnp.maximum(a + cjk, 0.0))
    h = jnp.concatenate(hs, axis=0)

    h = jnp.maximum(jnp.dot(h, w2_ref[...],
                            preferred_element_type=jnp.float32) + b2_ref[...], 0.0)
    msg = jnp.dot(h, w3_ref[...],
                  preferred_element_type=jnp.float32) + b3_ref[...]

    pf = msg[0:BLK, :]
    for k in range(1, K):
        pf = jnp.maximum(pf, msg[k * BLK:(k + 1) * BLK, :])

    rows = []
    for s in range(NUM_SLICES):
        sel = br == s
        rows.append(jnp.max(jnp.where(sel, pf, -_INF), axis=0, keepdims=True))
    blk_out = jnp.concatenate(rows, axis=0)

    @pl.when(i == 0)
    def _():
        out_ref[...] = jnp.full((NUM_SLICES, D_OUT), -_INF, jnp.float32)

    out_ref[...] = jnp.maximum(out_ref[...], blk_out)


def kernel(x, batch, W1, b1, W2, b2, W3, b3):
    batch_i = batch.astype(jnp.int32)
    xc = x.T
    br = batch_i[:, None]
    bc = batch_i[None, :]
    sl = jnp.arange(NUM_SLICES, dtype=jnp.int32)
    starts = jnp.searchsorted(batch_i, sl, side="left").astype(jnp.int32)
    ends = jnp.searchsorted(batch_i, sl, side="right").astype(jnp.int32)
    first_b = batch_i[0::BLK]
    last_b = batch_i[BLK - 1::BLK]
    clo = (starts[first_b] // CW).astype(jnp.int32)
    chi = ((ends[last_b] + CW - 1) // CW).astype(jnp.int32)
    wi0 = (W1[0] - W1[2])[None, :]
    wi1 = (W1[1] - W1[3])[None, :]
    wj0 = W1[2][None, :]
    wj1 = W1[3][None, :]

    smem = pl.BlockSpec(memory_space=pltpu.SMEM)
    full = lambda shape: pl.BlockSpec(shape, lambda i: (0, 0))
    out = pl.pallas_call(
        _body,
        grid=(NBLK,),
        in_specs=[
            smem, smem,
            pl.BlockSpec((BLK, 2), lambda i: (i, 0)),
            pl.BlockSpec((BLK, 1), lambda i: (i, 0)),
            full((2, N)),
            full((1, N)),
            full((1, 64)),
            full((1, 64)),
            full((1, 64)),
            full((1, 64)),
            full((1, 64)),
            full((64, 128)),
            full((1, 128)),
            full((128, 256)),
            full((1, 256)),
        ],
        out_specs=pl.BlockSpec((NUM_SLICES, D_OUT), lambda i: (0, 0)),
        out_shape=jax.ShapeDtypeStruct((NUM_SLICES, D_OUT), jnp.float32),
    )(clo, chi, x, br, xc, bc, wi0, wi1, wj0, wj1,
      b1[None, :], W2, b2[None, :], W3, b3[None, :])
    return out


# speedup vs baseline: 1.9128x; 1.2837x over previous
import jax
import jax.numpy as jnp
from jax import lax
from jax.experimental import pallas as pl
from jax.experimental.pallas import tpu as pltpu

N = 8192
NUM_SLICES = 16
K = 20
BLK = 256
CW = 512
NBLK = N // BLK
BESTW = 128
D_OUT = 256

_INF = float("inf")
_BIG_I = 2**30


def _body(clo_ref, chi_ref, xr_ref, br_ref, xc_ref, bc_ref,
          wi0_ref, wi1_ref, wj0_ref, wj1_ref,
          b1_ref, w2_ref, b2_ref, w3_ref, b3_ref, out_ref):
    i = pl.program_id(0)
    xr = xr_ref[...]
    br = br_ref[...]
    sq_r = jnp.sum(xr * xr, axis=1, keepdims=True)

    row_iota = lax.broadcasted_iota(jnp.int32, (BLK, CW), 0)
    col_iota = lax.broadcasted_iota(jnp.int32, (BLK, CW), 1)
    iota_c = lax.broadcasted_iota(
        jnp.int32, (BLK, CW), 1).astype(jnp.float32)
    iota_w = lax.broadcasted_iota(
        jnp.int32, (BLK, BESTW + CW), 1).astype(jnp.float32)

    def masked_d2(c):
        cs = c * CW
        xw = xc_ref[:, pl.ds(cs, CW)]
        bw = bc_ref[:, pl.ds(cs, CW)]
        xw0 = xw[0:1, :]
        xw1 = xw[1:2, :]
        sq_w = xw0 * xw0 + xw1 * xw1
        dot = jnp.dot(xr, xw, preferred_element_type=jnp.float32)
        d2 = (sq_r + sq_w) - 2.0 * dot
        same = br == bw
        d2 = jnp.where(same, d2, _INF)
        rowg = i * BLK + row_iota
        colg = cs + col_iota
        return jnp.where(rowg == colg, _INF, d2), xw0, xw1

    def topk_scan(wa, wx0, wx1, iota):
        ks, x0s, x1s = [], [], []
        for _ in range(K):
            m = jnp.min(wa, axis=1, keepdims=True)
            pos = jnp.min(jnp.where(wa == m, iota, _INF),
                          axis=1, keepdims=True)
            hit = iota == pos
            x0s.append(jnp.sum(jnp.where(hit, wx0, 0.0), axis=1, keepdims=True))
            x1s.append(jnp.sum(jnp.where(hit, wx1, 0.0), axis=1, keepdims=True))
            wa = jnp.where(hit, _INF, wa)
            ks.append(m)
        return ks, x0s, x1s

    def pack_best(ks, x0s, x1s):
        pad = jnp.full((BLK, BESTW - K), _INF, jnp.float32)
        zpad = jnp.zeros((BLK, BESTW - K), jnp.float32)
        return (jnp.concatenate(ks + [pad], axis=1),
                jnp.concatenate(x0s + [zpad], axis=1),
                jnp.concatenate(x1s + [zpad], axis=1))

    def chunk_body(c, carry):
        best_key, best_x0, best_x1 = carry
        d2, xw0, xw1 = masked_d2(c)
        wa = jnp.concatenate([best_key, d2], axis=1)
        wx0 = jnp.concatenate(
            [best_x0, jnp.broadcast_to(xw0, (BLK, CW))], axis=1)
        wx1 = jnp.concatenate(
            [best_x1, jnp.broadcast_to(xw1, (BLK, CW))], axis=1)
        return pack_best(*topk_scan(wa, wx0, wx1, iota_w))

    c0 = clo_ref[i]
    c1 = chi_ref[i]
    d2f, xw0f, xw1f = masked_d2(c0)
    init = pack_best(*topk_scan(d2f, xw0f, xw1f, iota_c))
    _, bx0, bx1 = lax.fori_loop(c0 + 1, c1, chunk_body, init)

    xr0 = xr[:, 0:1]
    xr1 = xr[:, 1:2]
    a = xr0 * wi0_ref[...] + xr1 * wi1_ref[...] + b1_ref[...]
    hs = []
    for k in range(K):
        cjk = bx0[:, k:k + 1] * wj0_ref[...] + bx1[:, k:k + 1] * wj1_ref[...]
        hs.append(jnp.maximum(a + cjk, 0.0))
    h = jnp.concatenate(hs, axis=0)

    h = jnp.maximum(jnp.dot(h, w2_ref[...],
                            preferred_element_type=jnp.float32) + b2_ref[...], 0.0)
    msg = jnp.dot(h, w3_ref[...],
                  preferred_element_type=jnp.float32) + b3_ref[...]

    pf = msg[0:BLK, :]
    for k in range(1, K):
        pf = jnp.maximum(pf, msg[k * BLK:(k + 1) * BLK, :])

    rows = []
    for s in range(NUM_SLICES):
        sel = br == s
        rows.append(jnp.max(jnp.where(sel, pf, -_INF), axis=0, keepdims=True))
    blk_out = jnp.concatenate(rows, axis=0)

    @pl.when(i == 0)
    def _():
        out_ref[...] = jnp.full((NUM_SLICES, D_OUT), -_INF, jnp.float32)

    out_ref[...] = jnp.maximum(out_ref[...], blk_out)


def kernel(x, batch, W1, b1, W2, b2, W3, b3):
    batch_i = batch.astype(jnp.int32)
    xc = x.T
    br = batch_i[:, None]
    bc = batch_i[None, :]
    sl = jnp.arange(NUM_SLICES, dtype=jnp.int32)
    starts = jnp.searchsorted(batch_i, sl, side="left").astype(jnp.int32)
    ends = jnp.searchsorted(batch_i, sl, side="right").astype(jnp.int32)
    first_b = batch_i[0::BLK]
    last_b = batch_i[BLK - 1::BLK]
    clo = (starts[first_b] // CW).astype(jnp.int32)
    chi = ((ends[last_b] + CW - 1) // CW).astype(jnp.int32)
    wi0 = (W1[0] - W1[2])[None, :]
    wi1 = (W1[1] - W1[3])[None, :]
    wj0 = W1[2][None, :]
    wj1 = W1[3][None, :]

    smem = pl.BlockSpec(memory_space=pltpu.SMEM)
    full = lambda shape: pl.BlockSpec(shape, lambda i: (0, 0))
    out = pl.pallas_call(
        _body,
        grid=(NBLK,),
        in_specs=[
            smem, smem,
            pl.BlockSpec((BLK, 2), lambda i: (i, 0)),
            pl.BlockSpec((BLK, 1), lambda i: (i, 0)),
            full((2, N)),
            full((1, N)),
            full((1, 64)),
            full((1, 64)),
            full((1, 64)),
            full((1, 64)),
            full((1, 64)),
            full((64, 128)),
            full((1, 128)),
            full((128, 256)),
            full((1, 256)),
        ],
        out_specs=pl.BlockSpec((NUM_SLICES, D_OUT), lambda i: (0, 0)),
        out_shape=jax.ShapeDtypeStruct((NUM_SLICES, D_OUT), jnp.float32),
    )(clo, chi, x, br, xc, bc, wi0, wi1, wj0, wj1,
      b1[None, :], W2, b2[None, :], W3, b3[None, :])
    return out
